# baseline (speedup 1.0000x reference)
import functools

import jax
import jax.numpy as jnp
from jax import lax
from jax.experimental import pallas as pl
from jax.experimental.pallas import tpu as pltpu
from jax.experimental.pallas import tpu_sc as plsc

N_SRC = 10000
N_DST = 2500
E = 160000
IN_CH = 256
HID = 512
H = 8
DH = 64
NT = 5
OUT_CH = 153

NC = 2
NS = 16
NW = NC * NS
EP = 163840
EPW = EP // NW
EB = 128
NB = EPW // EB
EPS = EP // NS
NBS = EPS // EB
HH = HID // 2
IC2 = IN_CH // 2

_MESH = plsc.VectorSubcoreMesh(core_axis_name="c", subcore_axis_name="s",
                               num_cores=NC, num_subcores=NS)


def _mm_bias_body(x_ref, w_ref, b_ref, o_ref):
    o_ref[...] = (
        jnp.dot(x_ref[...], w_ref[...], preferred_element_type=jnp.float32)
        + b_ref[...]
    )


def _matmul_bias(xm, wm, bv, bm_rows):
    M, K = xm.shape
    _, N = wm.shape
    return pl.pallas_call(
        _mm_bias_body,
        grid=(M // bm_rows,),
        in_specs=[
            pl.BlockSpec((bm_rows, K), lambda i: (i, 0)),
            pl.BlockSpec((K, N), lambda i: (0, 0)),
            pl.BlockSpec((1, N), lambda i: (0, 0)),
        ],
        out_specs=pl.BlockSpec((bm_rows, N), lambda i: (i, 0)),
        out_shape=jax.ShapeDtypeStruct((M, N), jnp.float32),
    )(xm, wm, bv.reshape(1, -1))


def _etype_body(x_ref, w_ref, o_ref):
    o_ref[0] = jnp.dot(x_ref[...], w_ref[0], preferred_element_type=jnp.float32)


def _fs_half_body(x_ref, w_ref, o_ref):
    o_ref[0, 0] = jnp.dot(x_ref[...], w_ref[0, 0],
                          preferred_element_type=jnp.float32)


def _fs_half_mm(xm, wg, bm_rows):
    M, K = xm.shape
    J = wg.shape[1]
    return pl.pallas_call(
        _fs_half_body,
        grid=(J, M // bm_rows, NC),
        in_specs=[
            pl.BlockSpec((bm_rows, K), lambda j, i, c: (i, 0)),
            pl.BlockSpec((1, 1, K, HH), lambda j, i, c: (c, j, 0, 0)),
        ],
        out_specs=pl.BlockSpec((1, 1, bm_rows, HH), lambda j, i, c: (c, j, i, 0)),
        out_shape=jax.ShapeDtypeStruct((NC, J, M, HH), jnp.float32),
    )(xm, wg)


def _etype_mm(xm, wg, bm_rows):
    M, K = xm.shape
    J, _, N = wg.shape
    return pl.pallas_call(
        _etype_body,
        grid=(J, M // bm_rows),
        in_specs=[
            pl.BlockSpec((bm_rows, K), lambda j, i: (i, 0)),
            pl.BlockSpec((1, K, N), lambda j, i: (j, 0, 0)),
        ],
        out_specs=pl.BlockSpec((1, bm_rows, N), lambda j, i: (j, i, 0)),
        out_shape=jax.ShapeDtypeStruct((J, M, N), jnp.float32),
    )(xm, wg)


def _denr_body(a_ref, o_ref):
    o_ref[...] = 1.0 / jnp.maximum(a_ref[0] + a_ref[1], 1e-9)


def _denr(denparts):
    return pl.pallas_call(
        _denr_body,
        out_shape=jax.ShapeDtypeStruct((NT * N_DST, 16), jnp.float32),
    )(denparts)


def _final_body(acc_ref, skip_ref, g1_ref, b1_ref, w1_ref, bm1_ref,
                g2_ref, b2_ref, w2_ref, bm2_ref, o_ref):
    t = jnp.concatenate([acc_ref[0], acc_ref[1]], axis=-1) + skip_ref[...]
    h = t * g1_ref[...] + b1_ref[...]
    h = jnp.where(h > 0, h, jnp.exp(h) - 1.0)
    h = jnp.dot(h, w1_ref[...], preferred_element_type=jnp.float32) + bm1_ref[...]
    h = jnp.maximum(h * g2_ref[...] + b2_ref[...], 0.0)
    o_ref[...] = jnp.dot(h, w2_ref[...], preferred_element_type=jnp.float32) + bm2_ref[...]


def _final(accparts, xskip, g1, b1, w1, bm1, g2, b2, w2, bm2):
    bm = 512
    grid = (N_DST + bm - 1) // bm
    row = lambda v: v.reshape(1, -1)
    return pl.pallas_call(
        _final_body,
        grid=(grid,),
        in_specs=[
            pl.BlockSpec((NC, bm, HH), lambda i: (0, i, 0)),
            pl.BlockSpec((bm, HID), lambda i: (i, 0)),
            pl.BlockSpec((1, HID), lambda i: (0, 0)),
            pl.BlockSpec((1, HID), lambda i: (0, 0)),
            pl.BlockSpec((HID, HID), lambda i: (0, 0)),
            pl.BlockSpec((1, HID), lambda i: (0, 0)),
            pl.BlockSpec((1, HID), lambda i: (0, 0)),
            pl.BlockSpec((1, HID), lambda i: (0, 0)),
            pl.BlockSpec((HID, OUT_CH), lambda i: (0, 0)),
            pl.BlockSpec((1, OUT_CH), lambda i: (0, 0)),
        ],
        out_specs=pl.BlockSpec((bm, OUT_CH), lambda i: (i, 0)),
        out_shape=jax.ShapeDtypeStruct((N_DST, OUT_CH), jnp.float32),
    )(accparts, xskip, row(g1), row(b1), w1, row(bm1), row(g2), row(b2), w2, row(bm2))


def _zero_shared(sh, zb, sid, per, n_last, nrows):
    n = jnp.where(sid < NS - 1, per, n_last)
    base = sid * per * 8

    def cp(q, c):
        pltpu.sync_copy(zb.at[pl.ds(0, 8)], sh.at[pl.ds(base + q * 8, 8)])
        return c

    lax.fori_loop(0, n, cp, 0)

    @pl.when(sid == NS - 1)
    def _():
        pltpu.sync_copy(zb.at[pl.ds(0, 4)], sh.at[pl.ds(nrows - 4, 4)])


def _zero_rows(zb, nrows, nchunks):
    z16 = jnp.zeros((16,), jnp.float32)

    def zr(e, c):
        for c2 in range(nchunks):
            zb[e, pl.ds(c2 * 16, 16)] = z16
        return c

    lax.fori_loop(0, nrows, zr, 0)


def _copy_out(sh, out_h, cid, sid, nrows):
    per = ((nrows // NS) // 8) * 8
    rbase = sid * per
    last = nrows - per * (NS - 1)

    @pl.when(sid < NS - 1)
    def _():
        pltpu.sync_copy(sh.at[pl.ds(rbase, per)], out_h.at[cid, pl.ds(rbase, per)])

    @pl.when(sid == NS - 1)
    def _():
        pltpu.sync_copy(sh.at[pl.ds(per * (NS - 1), last)],
                        out_h.at[cid, pl.ds(per * (NS - 1), last)])


@functools.partial(
    pl.kernel,
    out_type=jax.ShapeDtypeStruct((NC, N_DST, IC2), jnp.float32),
    mesh=_MESH,
    compiler_params=pltpu.CompilerParams(needs_layout_passes=False, use_tc_tiling_on_sc=False),
    scratch_types=[
        [pltpu.VMEM((EB,), jnp.int32)] * 2,
        [pltpu.VMEM((EB,), jnp.int32)] * 2,
        [pltpu.VMEM((EB,), jnp.int32)] * 2,
        pltpu.VMEM((N_SRC,), jnp.int32),
        pltpu.VMEM((N_SRC,), jnp.float32),
        [pltpu.VMEM((EB,), jnp.float32)] * 2,
        [pltpu.VMEM((EB, IC2), jnp.float32)] * 2,
        pltpu.VMEM_SHARED((N_DST, IC2), jnp.float32),
        [pltpu.SemaphoreType.DMA] * 2,
        [pltpu.SemaphoreType.DMA] * 2,
        pltpu.SemaphoreType.DMA,
    ],
)
def _temporal_sc(src_h, dst_h, et_h, yrs_h, alt_h, xl2_h, out_h,
                 sbufs, dbufs, tbufs, ybuf, abuf, avbufs, xlbufs, t_sh,
                 sem_m, sem_g, sem_s):
    cid = lax.axis_index("c")
    sid = lax.axis_index("s")

    _zero_rows(xlbufs[0], 20, IC2 // 16)
    _zero_shared(t_sh, xlbufs[0], sid, 20, 12, N_DST)
    pltpu.sync_copy(yrs_h, ybuf)
    pltpu.sync_copy(alt_h, abuf)
    plsc.subcore_barrier()

    lanes = lax.iota(jnp.int32, 16)
    base = sid * EPS

    def fire_meta(q, b):
        off = base + b * EB
        pltpu.async_copy(src_h.at[pl.ds(off, EB)], sbufs[q], sem_m[q])
        pltpu.async_copy(dst_h.at[pl.ds(off, EB)], dbufs[q], sem_m[q])
        pltpu.async_copy(et_h.at[pl.ds(off, EB)], tbufs[q], sem_m[q])

    def wait_meta(q, b):
        off = base + b * EB
        pltpu.make_async_copy(src_h.at[pl.ds(off, EB)], sbufs[q], sem_m[q]).wait()
        pltpu.make_async_copy(dst_h.at[pl.ds(off, EB)], dbufs[q], sem_m[q]).wait()
        pltpu.make_async_copy(et_h.at[pl.ds(off, EB)], tbufs[q], sem_m[q]).wait()

    def prep(q, b):
        off = base + b * EB

        def grp(g, c2):
            s16 = sbufs[q][pl.ds(g * 16, 16)]
            d16 = dbufs[q][pl.ds(g * 16, 16)]
            t16 = tbufs[q][pl.ds(g * 16, 16)]
            y1 = plsc.load_gather(ybuf, [s16])
            y2 = plsc.load_gather(ybuf, [d16])
            als = plsc.load_gather(abuf, [s16])
            gap = jnp.exp(-jnp.abs((y1 - y2).astype(jnp.float32)))
            a = als * gap
            a = jnp.where(a >= 0, a, 0.2 * a)
            ok = (t16 == 4) & ((off + g * 16 + lanes) < E)
            avbufs[q][pl.ds(g * 16, 16)] = jnp.where(ok, a, 0.0)
            sbufs[q][pl.ds(g * 16, 16)] = s16 * NC + cid
            return c2

        lax.fori_loop(0, EB // 16, grp, 0)
        pltpu.async_copy(xl2_h.at[sbufs[q]], xlbufs[q], sem_g[q])

    def wait_rows(q):
        pltpu.make_async_copy(xl2_h.at[sbufs[q]], xlbufs[q], sem_g[q]).wait()

    def scale(q):
        def edge(e, c2):
            av = plsc.load_gather(avbufs[q], [jnp.full((16,), 0, jnp.int32) + e])
            for c in range(IC2 // 16):
                sl = pl.ds(c * 16, 16)
                xlbufs[q][e, sl] = xlbufs[q][e, sl] * av
            return c2

        lax.fori_loop(0, EB, edge, 0)

    def scatter(q):
        pltpu.sync_copy(xlbufs[q], t_sh.at[dbufs[q]], add=True)

    def scatter_async(q):
        pltpu.async_copy(xlbufs[q], t_sh.at[dbufs[q]], sem_s, add=True)

    def wait_scatter(q):
        pltpu.make_async_copy(xlbufs[q], t_sh.at[dbufs[q]], sem_s).wait()

    fire_meta(0, 0)
    wait_meta(0, 0)
    prep(0, 0)
    fire_meta(1, 1)

    def pipe(i, carry):
        b = 2 * i
        wait_meta(1, b + 1)
        prep(1, b + 1)
        wait_rows(0)
        scale(0)
        scatter_async(0)

        wait_rows(1)
        scale(1)
        wait_scatter(0)

        @pl.when(b + 2 < NBS)
        def _():
            fire_meta(0, b + 2)
            wait_meta(0, b + 2)
            prep(0, b + 2)
        scatter(1)

        @pl.when(b + 3 < NBS)
        def _():
            fire_meta(1, b + 3)
        return carry

    lax.fori_loop(0, NBS // 2, pipe, 0)
    plsc.subcore_barrier()
    _copy_out(t_sh, out_h, cid, sid, N_DST)


@functools.partial(
    pl.kernel,
    out_type=jax.ShapeDtypeStruct((NC, NT * N_DST, 16), jnp.float32),
    mesh=_MESH,
    compiler_params=pltpu.CompilerParams(needs_layout_passes=False, use_tc_tiling_on_sc=False),
    scratch_types=[
        [pltpu.VMEM((EB,), jnp.int32)] * 2,
        [pltpu.VMEM((EB,), jnp.int32)] * 2,
        [pltpu.VMEM((EB,), jnp.int32)] * 2,
        [pltpu.VMEM((EB,), jnp.int32)] * 2,
        [pltpu.VMEM((EB,), jnp.int32)] * 2,
        [pltpu.VMEM((EB, 16), jnp.float32)] * 2,
        [pltpu.VMEM((EB, 16), jnp.float32)] * 2,
        pltpu.VMEM((H, EB), jnp.float32),
        pltpu.VMEM((EB, 16), jnp.float32),
        pltpu.VMEM_SHARED((NT * N_DST, 16), jnp.float32),
        [pltpu.SemaphoreType.DMA] * 2,
        [pltpu.SemaphoreType.DMA] * 2,
    ],
)
def _den_sc(src_h, dst_h, et_h, el_h, er_h, out_h,
            sbufs, dbufs, tbufs, gbufs, kbufs, elbufs, erbufs,
            wbuf, msgbuf, den_sh, sem_m, sem_g):
    cid = lax.axis_index("c")
    sid = lax.axis_index("s")
    wid = sid * NC + cid

    _zero_rows(msgbuf, 20, 1)
    _zero_shared(den_sh, msgbuf, sid, 97, 107, NT * N_DST)
    plsc.subcore_barrier()
    lanes = lax.iota(jnp.int32, 16)
    base = wid * EPW

    def fire_meta(q, b):
        off = base + b * EB
        pltpu.async_copy(src_h.at[pl.ds(off, EB)], sbufs[q], sem_m[q])
        pltpu.async_copy(dst_h.at[pl.ds(off, EB)], dbufs[q], sem_m[q])
        pltpu.async_copy(et_h.at[pl.ds(off, EB)], tbufs[q], sem_m[q])

    def wait_meta(q, b):
        off = base + b * EB
        pltpu.make_async_copy(src_h.at[pl.ds(off, EB)], sbufs[q], sem_m[q]).wait()
        pltpu.make_async_copy(dst_h.at[pl.ds(off, EB)], dbufs[q], sem_m[q]).wait()
        pltpu.make_async_copy(et_h.at[pl.ds(off, EB)], tbufs[q], sem_m[q]).wait()

    def prep(q):
        def mk(i, c2):
            s16 = sbufs[q][pl.ds(i * 16, 16)]
            d16 = dbufs[q][pl.ds(i * 16, 16)]
            t16 = tbufs[q][pl.ds(i * 16, 16)]
            gbufs[q][pl.ds(i * 16, 16)] = t16 * N_SRC + s16
            kbufs[q][pl.ds(i * 16, 16)] = t16 * N_DST + d16
            return c2

        lax.fori_loop(0, EB // 16, mk, 0)
        pltpu.async_copy(el_h.at[gbufs[q]], elbufs[q], sem_g[q])
        pltpu.async_copy(er_h.at[kbufs[q]], erbufs[q], sem_g[q])

    def wait_rows(q):
        pltpu.make_async_copy(el_h.at[gbufs[q]], elbufs[q], sem_g[q]).wait()
        pltpu.make_async_copy(er_h.at[kbufs[q]], erbufs[q], sem_g[q]).wait()

    def compute_scatter(q, b):
        off = base + b * EB

        def grp(g, c2):
            e16 = g * 16 + lanes
            ok = (off + e16) < E
            for h in range(H):
                hh = jnp.full((16,), h, jnp.int32)
                elh = plsc.load_gather(elbufs[q], [e16, hh])
                erh = plsc.load_gather(erbufs[q], [e16, hh])
                z = elh + erh
                z = jnp.where(z >= 0, z, 0.2 * z)
                wbuf[h, pl.ds(g * 16, 16)] = jnp.where(ok, jnp.exp(z), 0.0)
            return c2

        lax.fori_loop(0, EB // 16, grp, 0)

        def edge(e, c2):
            e0 = jnp.full((16,), 0, jnp.int32) + e
            rowv = plsc.load_gather(wbuf, [lanes & 7, e0])
            msgbuf[e, pl.ds(0, 16)] = jnp.where(lanes < 8, rowv, 0.0)
            return c2

        lax.fori_loop(0, EB, edge, 0)
        pltpu.sync_copy(msgbuf, den_sh.at[kbufs[q]], add=True)

    fire_meta(0, 0)
    wait_meta(0, 0)
    prep(0)
    fire_meta(1, 1)

    def pipe(i, carry):
        b = 2 * i
        wait_meta(1, b + 1)
        prep(1)
        wait_rows(0)
        compute_scatter(0, b)

        @pl.when(b + 2 < NB)
        def _():
            fire_meta(0, b + 2)

        @pl.when(b + 2 < NB)
        def _():
            wait_meta(0, b + 2)
            prep(0)
        wait_rows(1)
        compute_scatter(1, b + 1)

        @pl.when(b + 3 < NB)
        def _():
            fire_meta(1, b + 3)
        return carry

    lax.fori_loop(0, NB // 2, pipe, 0)
    plsc.subcore_barrier()
    _copy_out(den_sh, out_h, cid, sid, NT * N_DST)


@functools.partial(
    pl.kernel,
    out_type=jax.ShapeDtypeStruct((NC, N_DST, HH), jnp.float32),
    mesh=_MESH,
    compiler_params=pltpu.CompilerParams(needs_layout_passes=False, use_tc_tiling_on_sc=False),
    scratch_types=[
        [pltpu.VMEM((EB,), jnp.int32)] * 2,
        [pltpu.VMEM((EB,), jnp.int32)] * 2,
        [pltpu.VMEM((EB,), jnp.int32)] * 2,
        [pltpu.VMEM((EB,), jnp.int32)] * 2,
        [pltpu.VMEM((EB,), jnp.int32)] * 2,
        [pltpu.VMEM((EB,), jnp.int32)] * 2,
        [pltpu.VMEM((EB, 16), jnp.float32)] * 2,
        [pltpu.VMEM((EB, 16), jnp.float32)] * 2,
        [pltpu.VMEM((EB, 16), jnp.float32)] * 2,
        [pltpu.VMEM((EB, HH), jnp.float32)] * 2,
        pltpu.VMEM((H, EB), jnp.float32),
        pltpu.VMEM_SHARED((N_DST, HH), jnp.float32),
        [pltpu.SemaphoreType.DMA] * 2,
        [pltpu.SemaphoreType.DMA] * 2,
        pltpu.SemaphoreType.DMA,
    ],
)
def _gat_sc(src_h, dst_h, et_h, fs_h, el_h, er_h, dr_h, out_h,
            sbufs, dbufs, tbufs, gbufs, kbufs, obufs, elbufs, erbufs, drbufs,
            fsbufs, albuf, acc_sh, sem_m, sem_g, sem_s):
    cid = lax.axis_index("c")
    sid = lax.axis_index("s")
    wid = sid * NC + cid

    _zero_rows(fsbufs[0], 20, HH // 16)
    _zero_shared(acc_sh, fsbufs[0], sid, 20, 12, N_DST)
    plsc.subcore_barrier()
    lanes = lax.iota(jnp.int32, 16)
    base = sid * EPS

    def fire_meta(q, b):
        off = base + b * EB
        pltpu.async_copy(src_h.at[pl.ds(off, EB)], sbufs[q], sem_m[q])
        pltpu.async_copy(dst_h.at[pl.ds(off, EB)], dbufs[q], sem_m[q])
        pltpu.async_copy(et_h.at[pl.ds(off, EB)], tbufs[q], sem_m[q])

    def wait_meta(q, b):
        off = base + b * EB
        pltpu.make_async_copy(src_h.at[pl.ds(off, EB)], sbufs[q], sem_m[q]).wait()
        pltpu.make_async_copy(dst_h.at[pl.ds(off, EB)], dbufs[q], sem_m[q]).wait()
        pltpu.make_async_copy(et_h.at[pl.ds(off, EB)], tbufs[q], sem_m[q]).wait()

    def mk(q):
        def body(i, c2):
            s16 = sbufs[q][pl.ds(i * 16, 16)]
            d16 = dbufs[q][pl.ds(i * 16, 16)]
            t16 = tbufs[q][pl.ds(i * 16, 16)]
            fi = t16 * N_SRC + s16
            obufs[q][pl.ds(i * 16, 16)] = fi
            gbufs[q][pl.ds(i * 16, 16)] = fi + cid * (NT * N_SRC)
            kbufs[q][pl.ds(i * 16, 16)] = t16 * N_DST + d16
            return c2
        lax.fori_loop(0, EB // 16, body, 0)

    def fire_gathers(q):
        pltpu.async_copy(fs_h.at[gbufs[q]], fsbufs[q], sem_g[q])
        pltpu.async_copy(el_h.at[obufs[q]], elbufs[q], sem_g[q])
        pltpu.async_copy(er_h.at[kbufs[q]], erbufs[q], sem_g[q])
        pltpu.async_copy(dr_h.at[kbufs[q]], drbufs[q], sem_g[q])

    def wait_gathers(q):
        pltpu.make_async_copy(fs_h.at[gbufs[q]], fsbufs[q], sem_g[q]).wait()
        pltpu.make_async_copy(el_h.at[obufs[q]], elbufs[q], sem_g[q]).wait()
        pltpu.make_async_copy(er_h.at[kbufs[q]], erbufs[q], sem_g[q]).wait()
        pltpu.make_async_copy(dr_h.at[kbufs[q]], drbufs[q], sem_g[q]).wait()

    def compute(q, b):
        off = base + b * EB

        def grp(g, c2):
            e16 = g * 16 + lanes
            ok = (off + e16) < E
            for h in range(H // 2):
                hh = jnp.full((16,), h, jnp.int32) + cid * (H // 2)
                elh = plsc.load_gather(elbufs[q], [e16, hh])
                erh = plsc.load_gather(erbufs[q], [e16, hh])
                drh = plsc.load_gather(drbufs[q], [e16, hh])
                z = elh + erh
                z = jnp.where(z >= 0, z, 0.2 * z)
                albuf[h, pl.ds(g * 16, 16)] = jnp.where(ok, jnp.exp(z) * drh, 0.0)
            return c2

        lax.fori_loop(0, EB // 16, grp, 0)

        def edge(e, c2):
            e0 = jnp.full((16,), 0, jnp.int32) + e
            for h in range(H // 2):
                av = plsc.load_gather(albuf, [jnp.full((16,), h, jnp.int32), e0])
                for c in range(DH // 16):
                    sl = pl.ds(h * DH + c * 16, 16)
                    fsbufs[q][e, sl] = fsbufs[q][e, sl] * av
            return c2

        lax.fori_loop(0, EB, edge, 0)

    def scatter(q):
        pltpu.sync_copy(fsbufs[q], acc_sh.at[dbufs[q]], add=True)

    def scatter_async(q):
        pltpu.async_copy(fsbufs[q], acc_sh.at[dbufs[q]], sem_s, add=True)

    def wait_scatter(q):
        pltpu.make_async_copy(fsbufs[q], acc_sh.at[dbufs[q]], sem_s).wait()

    fire_meta(0, 0)
    wait_meta(0, 0)
    mk(0)
    fire_gathers(0)
    fire_meta(1, 1)

    def pipe(i, carry):
        b = 2 * i

        wait_meta(1, b + 1)
        mk(1)
        fire_gathers(1)
        wait_gathers(0)
        compute(0, b)
        scatter_async(0)

        wait_gathers(1)
        compute(1, b + 1)
        wait_scatter(0)

        @pl.when(b + 2 < NBS)
        def _():
            fire_meta(0, b + 2)
            wait_meta(0, b + 2)
            mk(0)
            fire_gathers(0)
        scatter(1)

        @pl.when(b + 3 < NBS)
        def _():
            fire_meta(1, b + 3)
        return carry

    lax.fori_loop(0, NBS // 2, pipe, 0)
    plsc.subcore_barrier()
    _copy_out(acc_sh, out_h, cid, sid, N_DST)


def kernel(x, edge_index, etype, years, n_dst, W_skip, b_skip, W_gat,
           attn_l, attn_r, b_gat, att_t, W_t, b_t, bn1_g, bn1_b,
           W_m1, b_m1, bnm_g, bnm_b, W_m2, b_m2):
    f32 = jnp.float32
    i32 = jnp.int32
    src = edge_index[0].astype(i32)
    dst = edge_index[1].astype(i32)
    et = etype.astype(i32)
    yrs = years.astype(i32)

    pad = EP - E
    srcp = jnp.concatenate([src, jnp.zeros((pad,), i32)])
    dstp = jnp.concatenate([dst, jnp.zeros((pad,), i32)])
    etp = jnp.concatenate([et, jnp.zeros((pad,), i32)])

    Wg4 = W_gat.reshape(NT, IN_CH, H, DH)
    WL = jnp.einsum("jchd,jhd->jch", Wg4, attn_l)
    WR = jnp.einsum("jchd,jhd->jch", Wg4, attn_r)
    zpad = jnp.zeros((NT, IN_CH, 8), f32)
    WLpad = jnp.concatenate([WL, zpad], axis=2)
    WRpad = jnp.concatenate([WR, zpad], axis=2)
    wt_att = W_t @ att_t[0]
    b_att = jnp.dot(b_t, att_t[0])
    delta = (jnp.asarray(n_dst) - N_DST).astype(f32)
    bskip_eff = b_skip + delta + b_gat.sum(0)
    c_bn = 1.0 / jnp.sqrt(1.0 + 1e-5)

    AUXA = 384
    WA = jnp.concatenate([W_t, wt_att[:, None], jnp.zeros((IN_CH, AUXA - 257), f32)], axis=1)
    bA = jnp.concatenate([b_t, b_att[None], jnp.zeros((AUXA - 257,), f32)])
    auxA = _matmul_bias(x, WA, bA, 2000)
    xl = auxA[:, :IN_CH]
    alt = auxA[:, IN_CH]

    xskip = _matmul_bias(x[:N_DST], W_skip, bskip_eff, N_DST)

    fs = _fs_half_mm(x, W_gat.reshape(NT, IN_CH, NC, HH).transpose(2, 0, 1, 3), 2000)
    fs2 = fs.reshape(NC * NT * N_SRC, HH)
    el = _etype_mm(x, WLpad, 2000)
    elflat = el.reshape(NT * N_SRC, 16)
    er03 = _etype_mm(x[:N_DST], WRpad[:4], N_DST)

    xl2 = xl.reshape(N_SRC * NC, IC2)
    tparts = _temporal_sc(srcp, dstp, etp, yrs, alt, xl2)
    er4 = _matmul_bias(
        jnp.concatenate([tparts[0], tparts[1]], axis=1),
        WRpad[4], jnp.zeros((16,), f32), N_DST)
    erflat = jnp.concatenate([er03, er4[None]], axis=0).reshape(NT * N_DST, 16)

    denparts = _den_sc(srcp, dstp, etp, elflat, erflat)
    denr = _denr(denparts)

    accparts = _gat_sc(srcp, dstp, etp, fs2, elflat, erflat, denr)

    return _final(accparts, xskip,
                  bn1_g * c_bn, bn1_b, W_m1, b_m1,
                  bnm_g * c_bn, bnm_b, W_m2, b_m2)


# speedup vs baseline: 1.0757x; 1.0757x over previous
import functools

import jax
import jax.numpy as jnp
from jax import lax
from jax.experimental import pallas as pl
from jax.experimental.pallas import tpu as pltpu
from jax.experimental.pallas import tpu_sc as plsc

N_SRC = 10000
N_DST = 2500
E = 160000
IN_CH = 256
HID = 512
H = 8
DH = 64
NT = 5
OUT_CH = 153

NC = 2
NS = 16
NW = NC * NS
EP = 163840
EPW = EP // NW
EB = 128
NB = EPW // EB
EPS = EP // NS
NBS = EPS // EB
HH = HID // 2
IC2 = IN_CH // 2

_MESH = plsc.VectorSubcoreMesh(core_axis_name="c", subcore_axis_name="s",
                               num_cores=NC, num_subcores=NS)


def _mm_bias_body(x_ref, w_ref, b_ref, o_ref):
    o_ref[...] = (
        jnp.dot(x_ref[...], w_ref[...], preferred_element_type=jnp.float32)
        + b_ref[...]
    )


def _matmul_bias(xm, wm, bv, bm_rows):
    M, K = xm.shape
    _, N = wm.shape
    return pl.pallas_call(
        _mm_bias_body,
        grid=(M // bm_rows,),
        in_specs=[
            pl.BlockSpec((bm_rows, K), lambda i: (i, 0)),
            pl.BlockSpec((K, N), lambda i: (0, 0)),
            pl.BlockSpec((1, N), lambda i: (0, 0)),
        ],
        out_specs=pl.BlockSpec((bm_rows, N), lambda i: (i, 0)),
        out_shape=jax.ShapeDtypeStruct((M, N), jnp.float32),
    )(xm, wm, bv.reshape(1, -1))


def _etype_body(x_ref, w_ref, o_ref):
    o_ref[0] = jnp.dot(x_ref[...], w_ref[0], preferred_element_type=jnp.float32)


def _fs_half_body(x_ref, w_ref, o_ref):
    o_ref[0, 0] = jnp.dot(x_ref[...], w_ref[0, 0],
                          preferred_element_type=jnp.float32)


def _fs_half_mm(xm, wg, bm_rows):
    M, K = xm.shape
    J = wg.shape[1]
    return pl.pallas_call(
        _fs_half_body,
        grid=(J, M // bm_rows, NC),
        in_specs=[
            pl.BlockSpec((bm_rows, K), lambda j, i, c: (i, 0)),
            pl.BlockSpec((1, 1, K, HH), lambda j, i, c: (c, j, 0, 0)),
        ],
        out_specs=pl.BlockSpec((1, 1, bm_rows, HH), lambda j, i, c: (c, j, i, 0)),
        out_shape=jax.ShapeDtypeStruct((NC, J, M, HH), jnp.float32),
    )(xm, wg)


def _etype_mm(xm, wg, bm_rows):
    M, K = xm.shape
    J, _, N = wg.shape
    return pl.pallas_call(
        _etype_body,
        grid=(J, M // bm_rows),
        in_specs=[
            pl.BlockSpec((bm_rows, K), lambda j, i: (i, 0)),
            pl.BlockSpec((1, K, N), lambda j, i: (j, 0, 0)),
        ],
        out_specs=pl.BlockSpec((1, bm_rows, N), lambda j, i: (j, i, 0)),
        out_shape=jax.ShapeDtypeStruct((J, M, N), jnp.float32),
    )(xm, wg)


def _denr_body(a_ref, o_ref):
    o_ref[...] = 1.0 / jnp.maximum(a_ref[0] + a_ref[1], 1e-9)


def _denr(denparts):
    return pl.pallas_call(
        _denr_body,
        out_shape=jax.ShapeDtypeStruct((NT * N_DST, 16), jnp.float32),
    )(denparts)


def _final_body(acc_ref, skip_ref, g1_ref, b1_ref, w1_ref, bm1_ref,
                g2_ref, b2_ref, w2_ref, bm2_ref, o_ref):
    t = jnp.concatenate([acc_ref[0], acc_ref[1]], axis=-1) + skip_ref[...]
    h = t * g1_ref[...] + b1_ref[...]
    h = jnp.where(h > 0, h, jnp.exp(h) - 1.0)
    h = jnp.dot(h, w1_ref[...], preferred_element_type=jnp.float32) + bm1_ref[...]
    h = jnp.maximum(h * g2_ref[...] + b2_ref[...], 0.0)
    o_ref[...] = jnp.dot(h, w2_ref[...], preferred_element_type=jnp.float32) + bm2_ref[...]


def _final(accparts, xskip, g1, b1, w1, bm1, g2, b2, w2, bm2):
    bm = 512
    grid = (N_DST + bm - 1) // bm
    row = lambda v: v.reshape(1, -1)
    return pl.pallas_call(
        _final_body,
        grid=(grid,),
        in_specs=[
            pl.BlockSpec((NC, bm, HH), lambda i: (0, i, 0)),
            pl.BlockSpec((bm, HID), lambda i: (i, 0)),
            pl.BlockSpec((1, HID), lambda i: (0, 0)),
            pl.BlockSpec((1, HID), lambda i: (0, 0)),
            pl.BlockSpec((HID, HID), lambda i: (0, 0)),
            pl.BlockSpec((1, HID), lambda i: (0, 0)),
            pl.BlockSpec((1, HID), lambda i: (0, 0)),
            pl.BlockSpec((1, HID), lambda i: (0, 0)),
            pl.BlockSpec((HID, OUT_CH), lambda i: (0, 0)),
            pl.BlockSpec((1, OUT_CH), lambda i: (0, 0)),
        ],
        out_specs=pl.BlockSpec((bm, OUT_CH), lambda i: (i, 0)),
        out_shape=jax.ShapeDtypeStruct((N_DST, OUT_CH), jnp.float32),
    )(accparts, xskip, row(g1), row(b1), w1, row(bm1), row(g2), row(b2), w2, row(bm2))


def _zero_shared(sh, zb, sid, per, n_last, nrows):
    n = jnp.where(sid < NS - 1, per, n_last)
    base = sid * per * 8

    def cp(q, c):
        pltpu.sync_copy(zb.at[pl.ds(0, 8)], sh.at[pl.ds(base + q * 8, 8)])
        return c

    lax.fori_loop(0, n, cp, 0)

    @pl.when(sid == NS - 1)
    def _():
        pltpu.sync_copy(zb.at[pl.ds(0, 4)], sh.at[pl.ds(nrows - 4, 4)])


def _zero_rows(zb, nrows, nchunks):
    z16 = jnp.zeros((16,), jnp.float32)

    def zr(e, c):
        for c2 in range(nchunks):
            zb[e, pl.ds(c2 * 16, 16)] = z16
        return c

    lax.fori_loop(0, nrows, zr, 0)


def _copy_out(sh, out_h, cid, sid, nrows):
    per = ((nrows // NS) // 8) * 8
    rbase = sid * per
    last = nrows - per * (NS - 1)

    @pl.when(sid < NS - 1)
    def _():
        pltpu.sync_copy(sh.at[pl.ds(rbase, per)], out_h.at[cid, pl.ds(rbase, per)])

    @pl.when(sid == NS - 1)
    def _():
        pltpu.sync_copy(sh.at[pl.ds(per * (NS - 1), last)],
                        out_h.at[cid, pl.ds(per * (NS - 1), last)])


@functools.partial(
    pl.kernel,
    out_type=jax.ShapeDtypeStruct((NC, N_DST, IC2), jnp.float32),
    mesh=_MESH,
    compiler_params=pltpu.CompilerParams(needs_layout_passes=False, use_tc_tiling_on_sc=False),
    scratch_types=[
        [pltpu.VMEM((EB,), jnp.int32)] * 2,
        [pltpu.VMEM((EB,), jnp.int32)] * 2,
        [pltpu.VMEM((EB,), jnp.int32)] * 2,
        pltpu.VMEM((N_SRC,), jnp.int32),
        pltpu.VMEM((N_SRC,), jnp.float32),
        [pltpu.VMEM((EB,), jnp.float32)] * 2,
        [pltpu.VMEM((EB, IC2), jnp.float32)] * 2,
        pltpu.VMEM_SHARED((N_DST, IC2), jnp.float32),
        [pltpu.SemaphoreType.DMA] * 2,
        [pltpu.SemaphoreType.DMA] * 2,
        pltpu.SemaphoreType.DMA,
    ],
)
def _temporal_sc(src_h, dst_h, et_h, yrs_h, alt_h, xl2_h, out_h,
                 sbufs, dbufs, tbufs, ybuf, abuf, avbufs, xlbufs, t_sh,
                 sem_m, sem_g, sem_s):
    cid = lax.axis_index("c")
    sid = lax.axis_index("s")

    _zero_rows(xlbufs[0], 20, IC2 // 16)
    _zero_shared(t_sh, xlbufs[0], sid, 20, 12, N_DST)
    pltpu.sync_copy(yrs_h, ybuf)
    pltpu.sync_copy(alt_h, abuf)
    plsc.subcore_barrier()

    lanes = lax.iota(jnp.int32, 16)
    base = sid * EPS

    def fire_meta(q, b):
        off = base + b * EB
        pltpu.async_copy(src_h.at[pl.ds(off, EB)], sbufs[q], sem_m[q])
        pltpu.async_copy(dst_h.at[pl.ds(off, EB)], dbufs[q], sem_m[q])
        pltpu.async_copy(et_h.at[pl.ds(off, EB)], tbufs[q], sem_m[q])

    def wait_meta(q, b):
        off = base + b * EB
        pltpu.make_async_copy(src_h.at[pl.ds(off, EB)], sbufs[q], sem_m[q]).wait()
        pltpu.make_async_copy(dst_h.at[pl.ds(off, EB)], dbufs[q], sem_m[q]).wait()
        pltpu.make_async_copy(et_h.at[pl.ds(off, EB)], tbufs[q], sem_m[q]).wait()

    def prep(q, b):
        off = base + b * EB

        def grp(g, c2):
            s16 = sbufs[q][pl.ds(g * 16, 16)]
            d16 = dbufs[q][pl.ds(g * 16, 16)]
            t16 = tbufs[q][pl.ds(g * 16, 16)]
            y1 = plsc.load_gather(ybuf, [s16])
            y2 = plsc.load_gather(ybuf, [d16])
            als = plsc.load_gather(abuf, [s16])
            gap = jnp.exp(-jnp.abs((y1 - y2).astype(jnp.float32)))
            a = als * gap
            a = jnp.where(a >= 0, a, 0.2 * a)
            ok = (t16 == 4) & ((off + g * 16 + lanes) < E)
            avbufs[q][pl.ds(g * 16, 16)] = jnp.where(ok, a, 0.0)
            sbufs[q][pl.ds(g * 16, 16)] = s16 * NC + cid
            return c2

        lax.fori_loop(0, EB // 16, grp, 0)
        pltpu.async_copy(xl2_h.at[sbufs[q]], xlbufs[q], sem_g[q])

    def wait_rows(q):
        pltpu.make_async_copy(xl2_h.at[sbufs[q]], xlbufs[q], sem_g[q]).wait()

    def scale(q):
        def edge(e, c2):
            av = plsc.load_gather(avbufs[q], [jnp.full((16,), 0, jnp.int32) + e])
            for c in range(IC2 // 16):
                sl = pl.ds(c * 16, 16)
                xlbufs[q][e, sl] = xlbufs[q][e, sl] * av
            return c2

        lax.fori_loop(0, EB, edge, 0)

    def scatter(q):
        pltpu.sync_copy(xlbufs[q], t_sh.at[dbufs[q]], add=True)

    def scatter_async(q):
        pltpu.async_copy(xlbufs[q], t_sh.at[dbufs[q]], sem_s, add=True)

    def wait_scatter(q):
        pltpu.make_async_copy(xlbufs[q], t_sh.at[dbufs[q]], sem_s).wait()

    fire_meta(0, 0)
    wait_meta(0, 0)
    prep(0, 0)
    fire_meta(1, 1)

    def pipe(i, carry):
        b = 2 * i
        wait_meta(1, b + 1)
        prep(1, b + 1)
        wait_rows(0)
        scale(0)
        scatter(0)

        @pl.when(b + 2 < NBS)
        def _():
            fire_meta(0, b + 2)

        @pl.when(b + 2 < NBS)
        def _():
            wait_meta(0, b + 2)
            prep(0, b + 2)
        wait_rows(1)
        scale(1)
        scatter(1)

        @pl.when(b + 3 < NBS)
        def _():
            fire_meta(1, b + 3)
        return carry

    lax.fori_loop(0, NBS // 2, pipe, 0)
    plsc.subcore_barrier()
    _copy_out(t_sh, out_h, cid, sid, N_DST)


@functools.partial(
    pl.kernel,
    out_type=jax.ShapeDtypeStruct((NC, NT * N_DST, 16), jnp.float32),
    mesh=_MESH,
    compiler_params=pltpu.CompilerParams(needs_layout_passes=False, use_tc_tiling_on_sc=False),
    scratch_types=[
        [pltpu.VMEM((EB,), jnp.int32)] * 2,
        [pltpu.VMEM((EB,), jnp.int32)] * 2,
        [pltpu.VMEM((EB,), jnp.int32)] * 2,
        [pltpu.VMEM((EB,), jnp.int32)] * 2,
        [pltpu.VMEM((EB,), jnp.int32)] * 2,
        [pltpu.VMEM((EB, 16), jnp.float32)] * 2,
        [pltpu.VMEM((EB, 16), jnp.float32)] * 2,
        pltpu.VMEM((H, EB), jnp.float32),
        pltpu.VMEM((EB, 16), jnp.float32),
        pltpu.VMEM_SHARED((NT * N_DST, 16), jnp.float32),
        [pltpu.SemaphoreType.DMA] * 2,
        [pltpu.SemaphoreType.DMA] * 2,
    ],
)
def _den_sc(src_h, dst_h, et_h, el_h, er_h, out_h,
            sbufs, dbufs, tbufs, gbufs, kbufs, elbufs, erbufs,
            wbuf, msgbuf, den_sh, sem_m, sem_g):
    cid = lax.axis_index("c")
    sid = lax.axis_index("s")
    wid = sid * NC + cid

    _zero_rows(msgbuf, 20, 1)
    _zero_shared(den_sh, msgbuf, sid, 97, 107, NT * N_DST)
    plsc.subcore_barrier()
    lanes = lax.iota(jnp.int32, 16)
    base = wid * EPW

    def fire_meta(q, b):
        off = base + b * EB
        pltpu.async_copy(src_h.at[pl.ds(off, EB)], sbufs[q], sem_m[q])
        pltpu.async_copy(dst_h.at[pl.ds(off, EB)], dbufs[q], sem_m[q])
        pltpu.async_copy(et_h.at[pl.ds(off, EB)], tbufs[q], sem_m[q])

    def wait_meta(q, b):
        off = base + b * EB
        pltpu.make_async_copy(src_h.at[pl.ds(off, EB)], sbufs[q], sem_m[q]).wait()
        pltpu.make_async_copy(dst_h.at[pl.ds(off, EB)], dbufs[q], sem_m[q]).wait()
        pltpu.make_async_copy(et_h.at[pl.ds(off, EB)], tbufs[q], sem_m[q]).wait()

    def prep(q):
        def mk(i, c2):
            s16 = sbufs[q][pl.ds(i * 16, 16)]
            d16 = dbufs[q][pl.ds(i * 16, 16)]
            t16 = tbufs[q][pl.ds(i * 16, 16)]
            gbufs[q][pl.ds(i * 16, 16)] = t16 * N_SRC + s16
            kbufs[q][pl.ds(i * 16, 16)] = t16 * N_DST + d16
            return c2

        lax.fori_loop(0, EB // 16, mk, 0)
        pltpu.async_copy(el_h.at[gbufs[q]], elbufs[q], sem_g[q])
        pltpu.async_copy(er_h.at[kbufs[q]], erbufs[q], sem_g[q])

    def wait_rows(q):
        pltpu.make_async_copy(el_h.at[gbufs[q]], elbufs[q], sem_g[q]).wait()
        pltpu.make_async_copy(er_h.at[kbufs[q]], erbufs[q], sem_g[q]).wait()

    def compute_scatter(q, b):
        off = base + b * EB

        def grp(g, c2):
            e16 = g * 16 + lanes
            ok = (off + e16) < E
            for h in range(H):
                hh = jnp.full((16,), h, jnp.int32)
                elh = plsc.load_gather(elbufs[q], [e16, hh])
                erh = plsc.load_gather(erbufs[q], [e16, hh])
                z = elh + erh
                z = jnp.where(z >= 0, z, 0.2 * z)
                wbuf[h, pl.ds(g * 16, 16)] = jnp.where(ok, jnp.exp(z), 0.0)
            return c2

        lax.fori_loop(0, EB // 16, grp, 0)

        def edge(e, c2):
            e0 = jnp.full((16,), 0, jnp.int32) + e
            rowv = plsc.load_gather(wbuf, [lanes & 7, e0])
            msgbuf[e, pl.ds(0, 16)] = jnp.where(lanes < 8, rowv, 0.0)
            return c2

        lax.fori_loop(0, EB, edge, 0)
        pltpu.sync_copy(msgbuf, den_sh.at[kbufs[q]], add=True)

    fire_meta(0, 0)
    wait_meta(0, 0)
    prep(0)
    fire_meta(1, 1)

    def pipe(i, carry):
        b = 2 * i
        wait_meta(1, b + 1)
        prep(1)
        wait_rows(0)
        compute_scatter(0, b)

        @pl.when(b + 2 < NB)
        def _():
            fire_meta(0, b + 2)

        @pl.when(b + 2 < NB)
        def _():
            wait_meta(0, b + 2)
            prep(0)
        wait_rows(1)
        compute_scatter(1, b + 1)

        @pl.when(b + 3 < NB)
        def _():
            fire_meta(1, b + 3)
        return carry

    lax.fori_loop(0, NB // 2, pipe, 0)
    plsc.subcore_barrier()
    _copy_out(den_sh, out_h, cid, sid, NT * N_DST)


@functools.partial(
    pl.kernel,
    out_type=jax.ShapeDtypeStruct((NC, N_DST, HH), jnp.float32),
    mesh=_MESH,
    compiler_params=pltpu.CompilerParams(needs_layout_passes=False, use_tc_tiling_on_sc=False),
    scratch_types=[
        [pltpu.VMEM((EB,), jnp.int32)] * 2,
        [pltpu.VMEM((EB,), jnp.int32)] * 2,
        [pltpu.VMEM((EB,), jnp.int32)] * 2,
        [pltpu.VMEM((EB,), jnp.int32)] * 2,
        [pltpu.VMEM((EB,), jnp.int32)] * 2,
        [pltpu.VMEM((EB,), jnp.int32)] * 2,
        [pltpu.VMEM((EB, 16), jnp.float32)] * 2,
        [pltpu.VMEM((EB, 16), jnp.float32)] * 2,
        [pltpu.VMEM((EB, 16), jnp.float32)] * 2,
        [pltpu.VMEM((EB, HH), jnp.float32)] * 2,
        pltpu.VMEM((H, EB), jnp.float32),
        pltpu.VMEM_SHARED((N_DST, HH), jnp.float32),
        [pltpu.SemaphoreType.DMA] * 2,
        [pltpu.SemaphoreType.DMA] * 2,
        pltpu.SemaphoreType.DMA,
    ],
)
def _gat_sc(src_h, dst_h, et_h, fs_h, el_h, er_h, dr_h, out_h,
            sbufs, dbufs, tbufs, gbufs, kbufs, obufs, elbufs, erbufs, drbufs,
            fsbufs, albuf, acc_sh, sem_m, sem_g, sem_s):
    cid = lax.axis_index("c")
    sid = lax.axis_index("s")
    wid = sid * NC + cid

    _zero_rows(fsbufs[0], 20, HH // 16)
    _zero_shared(acc_sh, fsbufs[0], sid, 20, 12, N_DST)
    plsc.subcore_barrier()
    lanes = lax.iota(jnp.int32, 16)
    base = sid * EPS

    def fire_meta(q, b):
        off = base + b * EB
        pltpu.async_copy(src_h.at[pl.ds(off, EB)], sbufs[q], sem_m[q])
        pltpu.async_copy(dst_h.at[pl.ds(off, EB)], dbufs[q], sem_m[q])
        pltpu.async_copy(et_h.at[pl.ds(off, EB)], tbufs[q], sem_m[q])

    def wait_meta(q, b):
        off = base + b * EB
        pltpu.make_async_copy(src_h.at[pl.ds(off, EB)], sbufs[q], sem_m[q]).wait()
        pltpu.make_async_copy(dst_h.at[pl.ds(off, EB)], dbufs[q], sem_m[q]).wait()
        pltpu.make_async_copy(et_h.at[pl.ds(off, EB)], tbufs[q], sem_m[q]).wait()

    def mk(q):
        def body(i, c2):
            s16 = sbufs[q][pl.ds(i * 16, 16)]
            d16 = dbufs[q][pl.ds(i * 16, 16)]
            t16 = tbufs[q][pl.ds(i * 16, 16)]
            fi = t16 * N_SRC + s16
            obufs[q][pl.ds(i * 16, 16)] = fi
            gbufs[q][pl.ds(i * 16, 16)] = fi + cid * (NT * N_SRC)
            kbufs[q][pl.ds(i * 16, 16)] = t16 * N_DST + d16
            return c2
        lax.fori_loop(0, EB // 16, body, 0)

    def fire_gathers(q):
        pltpu.async_copy(fs_h.at[gbufs[q]], fsbufs[q], sem_g[q])
        pltpu.async_copy(el_h.at[obufs[q]], elbufs[q], sem_g[q])
        pltpu.async_copy(er_h.at[kbufs[q]], erbufs[q], sem_g[q])
        pltpu.async_copy(dr_h.at[kbufs[q]], drbufs[q], sem_g[q])

    def wait_gathers(q):
        pltpu.make_async_copy(fs_h.at[gbufs[q]], fsbufs[q], sem_g[q]).wait()
        pltpu.make_async_copy(el_h.at[obufs[q]], elbufs[q], sem_g[q]).wait()
        pltpu.make_async_copy(er_h.at[kbufs[q]], erbufs[q], sem_g[q]).wait()
        pltpu.make_async_copy(dr_h.at[kbufs[q]], drbufs[q], sem_g[q]).wait()

    def compute(q, b):
        off = base + b * EB

        def grp(g, c2):
            e16 = g * 16 + lanes
            ok = (off + e16) < E
            for h in range(H // 2):
                hh = jnp.full((16,), h, jnp.int32) + cid * (H // 2)
                elh = plsc.load_gather(elbufs[q], [e16, hh])
                erh = plsc.load_gather(erbufs[q], [e16, hh])
                drh = plsc.load_gather(drbufs[q], [e16, hh])
                z = elh + erh
                z = jnp.where(z >= 0, z, 0.2 * z)
                albuf[h, pl.ds(g * 16, 16)] = jnp.where(ok, jnp.exp(z) * drh, 0.0)
            return c2

        lax.fori_loop(0, EB // 16, grp, 0)

        def edge(e, c2):
            e0 = jnp.full((16,), 0, jnp.int32) + e
            for h in range(H // 2):
                av = plsc.load_gather(albuf, [jnp.full((16,), h, jnp.int32), e0])
                for c in range(DH // 16):
                    sl = pl.ds(h * DH + c * 16, 16)
                    fsbufs[q][e, sl] = fsbufs[q][e, sl] * av
            return c2

        lax.fori_loop(0, EB, edge, 0)

    def scatter(q):
        pltpu.sync_copy(fsbufs[q], acc_sh.at[dbufs[q]], add=True)

    def scatter_async(q):
        pltpu.async_copy(fsbufs[q], acc_sh.at[dbufs[q]], sem_s, add=True)

    def wait_scatter(q):
        pltpu.make_async_copy(fsbufs[q], acc_sh.at[dbufs[q]], sem_s).wait()

    fire_meta(0, 0)
    wait_meta(0, 0)
    mk(0)
    fire_gathers(0)
    fire_meta(1, 1)

    def pipe(i, carry):
        b = 2 * i

        wait_meta(1, b + 1)
        mk(1)
        fire_gathers(1)
        wait_gathers(0)
        compute(0, b)
        scatter(0)

        @pl.when(b + 2 < NBS)
        def _():
            fire_meta(0, b + 2)

        @pl.when(b + 2 < NBS)
        def _():
            wait_meta(0, b + 2)
            mk(0)
            fire_gathers(0)
        wait_gathers(1)
        compute(1, b + 1)
        scatter(1)

        @pl.when(b + 3 < NBS)
        def _():
            fire_meta(1, b + 3)
        return carry

    lax.fori_loop(0, NBS // 2, pipe, 0)
    plsc.subcore_barrier()
    _copy_out(acc_sh, out_h, cid, sid, N_DST)


def kernel(x, edge_index, etype, years, n_dst, W_skip, b_skip, W_gat,
           attn_l, attn_r, b_gat, att_t, W_t, b_t, bn1_g, bn1_b,
           W_m1, b_m1, bnm_g, bnm_b, W_m2, b_m2):
    f32 = jnp.float32
    i32 = jnp.int32
    src = edge_index[0].astype(i32)
    dst = edge_index[1].astype(i32)
    et = etype.astype(i32)
    yrs = years.astype(i32)

    pad = EP - E
    srcp = jnp.concatenate([src, jnp.zeros((pad,), i32)])
    dstp = jnp.concatenate([dst, jnp.zeros((pad,), i32)])
    etp = jnp.concatenate([et, jnp.zeros((pad,), i32)])

    Wg4 = W_gat.reshape(NT, IN_CH, H, DH)
    WL = jnp.einsum("jchd,jhd->jch", Wg4, attn_l)
    WR = jnp.einsum("jchd,jhd->jch", Wg4, attn_r)
    zpad = jnp.zeros((NT, IN_CH, 8), f32)
    WLpad = jnp.concatenate([WL, zpad], axis=2)
    WRpad = jnp.concatenate([WR, zpad], axis=2)
    wt_att = W_t @ att_t[0]
    b_att = jnp.dot(b_t, att_t[0])
    delta = (jnp.asarray(n_dst) - N_DST).astype(f32)
    bskip_eff = b_skip + delta + b_gat.sum(0)
    c_bn = 1.0 / jnp.sqrt(1.0 + 1e-5)

    AUXA = 384
    WA = jnp.concatenate([W_t, wt_att[:, None], jnp.zeros((IN_CH, AUXA - 257), f32)], axis=1)
    bA = jnp.concatenate([b_t, b_att[None], jnp.zeros((AUXA - 257,), f32)])
    auxA = _matmul_bias(x, WA, bA, 2000)
    xl = auxA[:, :IN_CH]
    alt = auxA[:, IN_CH]

    xskip = _matmul_bias(x[:N_DST], W_skip, bskip_eff, N_DST)

    fs = _fs_half_mm(x, W_gat.reshape(NT, IN_CH, NC, HH).transpose(2, 0, 1, 3), 2000)
    fs2 = fs.reshape(NC * NT * N_SRC, HH)
    el = _etype_mm(x, WLpad, 2000)
    elflat = el.reshape(NT * N_SRC, 16)
    er03 = _etype_mm(x[:N_DST], WRpad[:4], N_DST)

    xl2 = xl.reshape(N_SRC * NC, IC2)
    tparts = _temporal_sc(srcp, dstp, etp, yrs, alt, xl2)
    er4 = _matmul_bias(
        jnp.concatenate([tparts[0], tparts[1]], axis=1),
        WRpad[4], jnp.zeros((16,), f32), N_DST)
    erflat = jnp.concatenate([er03, er4[None]], axis=0).reshape(NT * N_DST, 16)

    denparts = _den_sc(srcp, dstp, etp, elflat, erflat)
    denr = _denr(denparts)

    accparts = _gat_sc(srcp, dstp, etp, fs2, elflat, erflat, denr)

    return _final(accparts, xskip,
                  bn1_g * c_bn, bn1_b, W_m1, b_m1,
                  bnm_g * c_bn, bnm_b, W_m2, b_m2)


# speedup vs baseline: 1.0762x; 1.0005x over previous
import functools

import jax
import jax.numpy as jnp
from jax import lax
from jax.experimental import pallas as pl
from jax.experimental.pallas import tpu as pltpu
from jax.experimental.pallas import tpu_sc as plsc

N_SRC = 10000
N_DST = 2500
E = 160000
IN_CH = 256
HID = 512
H = 8
DH = 64
NT = 5
OUT_CH = 153

NC = 2
NS = 16
NW = NC * NS
EP = 163840
EPW = EP // NW
EB = 128
NB = EPW // EB
EPS = EP // NS
NBS = EPS // EB
HH = HID // 2
IC2 = IN_CH // 2

_MESH = plsc.VectorSubcoreMesh(core_axis_name="c", subcore_axis_name="s",
                               num_cores=NC, num_subcores=NS)


def _mm_bias_body(x_ref, w_ref, b_ref, o_ref):
    o_ref[...] = (
        jnp.dot(x_ref[...], w_ref[...], preferred_element_type=jnp.float32)
        + b_ref[...]
    )


def _matmul_bias(xm, wm, bv, bm_rows):
    M, K = xm.shape
    _, N = wm.shape
    return pl.pallas_call(
        _mm_bias_body,
        grid=(M // bm_rows,),
        in_specs=[
            pl.BlockSpec((bm_rows, K), lambda i: (i, 0)),
            pl.BlockSpec((K, N), lambda i: (0, 0)),
            pl.BlockSpec((1, N), lambda i: (0, 0)),
        ],
        out_specs=pl.BlockSpec((bm_rows, N), lambda i: (i, 0)),
        out_shape=jax.ShapeDtypeStruct((M, N), jnp.float32),
    )(xm, wm, bv.reshape(1, -1))


def _etype_body(x_ref, w_ref, o_ref):
    o_ref[0] = jnp.dot(x_ref[...], w_ref[0], preferred_element_type=jnp.float32)


def _fs_half_body(x_ref, w_ref, o_ref):
    o_ref[0, 0] = jnp.dot(x_ref[...], w_ref[0, 0],
                          preferred_element_type=jnp.float32)


def _fs_half_mm(xm, wg, bm_rows):
    M, K = xm.shape
    J = wg.shape[1]
    return pl.pallas_call(
        _fs_half_body,
        grid=(J, M // bm_rows, NC),
        in_specs=[
            pl.BlockSpec((bm_rows, K), lambda j, i, c: (i, 0)),
            pl.BlockSpec((1, 1, K, HH), lambda j, i, c: (c, j, 0, 0)),
        ],
        out_specs=pl.BlockSpec((1, 1, bm_rows, HH), lambda j, i, c: (c, j, i, 0)),
        out_shape=jax.ShapeDtypeStruct((NC, J, M, HH), jnp.float32),
    )(xm, wg)


def _etype_mm(xm, wg, bm_rows):
    M, K = xm.shape
    J, _, N = wg.shape
    return pl.pallas_call(
        _etype_body,
        grid=(J, M // bm_rows),
        in_specs=[
            pl.BlockSpec((bm_rows, K), lambda j, i: (i, 0)),
            pl.BlockSpec((1, K, N), lambda j, i: (j, 0, 0)),
        ],
        out_specs=pl.BlockSpec((1, bm_rows, N), lambda j, i: (j, i, 0)),
        out_shape=jax.ShapeDtypeStruct((J, M, N), jnp.float32),
    )(xm, wg)


def _denr_body(a_ref, o_ref):
    o_ref[...] = 1.0 / jnp.maximum(a_ref[0] + a_ref[1], 1e-9)


def _denr(denparts):
    return pl.pallas_call(
        _denr_body,
        out_shape=jax.ShapeDtypeStruct((NT * N_DST, 16), jnp.float32),
    )(denparts)


def _final_body(acc_ref, skip_ref, g1_ref, b1_ref, w1_ref, bm1_ref,
                g2_ref, b2_ref, w2_ref, bm2_ref, o_ref):
    t = jnp.concatenate([acc_ref[0], acc_ref[1]], axis=-1) + skip_ref[...]
    h = t * g1_ref[...] + b1_ref[...]
    h = jnp.where(h > 0, h, jnp.exp(h) - 1.0)
    h = jnp.dot(h, w1_ref[...], preferred_element_type=jnp.float32) + bm1_ref[...]
    h = jnp.maximum(h * g2_ref[...] + b2_ref[...], 0.0)
    o_ref[...] = jnp.dot(h, w2_ref[...], preferred_element_type=jnp.float32) + bm2_ref[...]


def _final(accparts, xskip, g1, b1, w1, bm1, g2, b2, w2, bm2):
    bm = 512
    grid = (N_DST + bm - 1) // bm
    row = lambda v: v.reshape(1, -1)
    return pl.pallas_call(
        _final_body,
        grid=(grid,),
        in_specs=[
            pl.BlockSpec((NC, bm, HH), lambda i: (0, i, 0)),
            pl.BlockSpec((bm, HID), lambda i: (i, 0)),
            pl.BlockSpec((1, HID), lambda i: (0, 0)),
            pl.BlockSpec((1, HID), lambda i: (0, 0)),
            pl.BlockSpec((HID, HID), lambda i: (0, 0)),
            pl.BlockSpec((1, HID), lambda i: (0, 0)),
            pl.BlockSpec((1, HID), lambda i: (0, 0)),
            pl.BlockSpec((1, HID), lambda i: (0, 0)),
            pl.BlockSpec((HID, OUT_CH), lambda i: (0, 0)),
            pl.BlockSpec((1, OUT_CH), lambda i: (0, 0)),
        ],
        out_specs=pl.BlockSpec((bm, OUT_CH), lambda i: (i, 0)),
        out_shape=jax.ShapeDtypeStruct((N_DST, OUT_CH), jnp.float32),
    )(accparts, xskip, row(g1), row(b1), w1, row(bm1), row(g2), row(b2), w2, row(bm2))


def _zero_shared(sh, zb, sid, per, n_last, nrows):
    n = jnp.where(sid < NS - 1, per, n_last)
    base = sid * per * 8

    def cp(q, c):
        pltpu.sync_copy(zb.at[pl.ds(0, 8)], sh.at[pl.ds(base + q * 8, 8)])
        return c

    lax.fori_loop(0, n, cp, 0)

    @pl.when(sid == NS - 1)
    def _():
        pltpu.sync_copy(zb.at[pl.ds(0, 4)], sh.at[pl.ds(nrows - 4, 4)])


def _zero_rows(zb, nrows, nchunks):
    z16 = jnp.zeros((16,), jnp.float32)

    def zr(e, c):
        for c2 in range(nchunks):
            zb[e, pl.ds(c2 * 16, 16)] = z16
        return c

    lax.fori_loop(0, nrows, zr, 0)


def _copy_out(sh, out_h, cid, sid, nrows):
    per = ((nrows // NS) // 8) * 8
    rbase = sid * per
    last = nrows - per * (NS - 1)

    @pl.when(sid < NS - 1)
    def _():
        pltpu.sync_copy(sh.at[pl.ds(rbase, per)], out_h.at[cid, pl.ds(rbase, per)])

    @pl.when(sid == NS - 1)
    def _():
        pltpu.sync_copy(sh.at[pl.ds(per * (NS - 1), last)],
                        out_h.at[cid, pl.ds(per * (NS - 1), last)])


@functools.partial(
    pl.kernel,
    out_type=jax.ShapeDtypeStruct((NC, N_DST, IC2), jnp.float32),
    mesh=_MESH,
    compiler_params=pltpu.CompilerParams(needs_layout_passes=False, use_tc_tiling_on_sc=False),
    scratch_types=[
        [pltpu.VMEM((EB,), jnp.int32)] * 2,
        [pltpu.VMEM((EB,), jnp.int32)] * 2,
        [pltpu.VMEM((EB,), jnp.int32)] * 2,
        pltpu.VMEM((N_SRC,), jnp.int32),
        pltpu.VMEM((N_SRC,), jnp.float32),
        [pltpu.VMEM((EB,), jnp.float32)] * 2,
        [pltpu.VMEM((EB, IC2), jnp.float32)] * 2,
        pltpu.VMEM_SHARED((N_DST, IC2), jnp.float32),
        [pltpu.SemaphoreType.DMA] * 2,
        [pltpu.SemaphoreType.DMA] * 2,
    ],
)
def _temporal_sc(src_h, dst_h, et_h, yrs_h, alt_h, xl2_h, out_h,
                 sbufs, dbufs, tbufs, ybuf, abuf, avbufs, xlbufs, t_sh,
                 sem_m, sem_g):
    cid = lax.axis_index("c")
    sid = lax.axis_index("s")

    _zero_rows(xlbufs[0], 20, IC2 // 16)
    _zero_shared(t_sh, xlbufs[0], sid, 20, 12, N_DST)
    pltpu.sync_copy(yrs_h, ybuf)
    pltpu.sync_copy(alt_h, abuf)
    plsc.subcore_barrier()

    lanes = lax.iota(jnp.int32, 16)
    base = sid * EPS

    def fire_meta(q, b):
        off = base + b * EB
        pltpu.async_copy(src_h.at[pl.ds(off, EB)], sbufs[q], sem_m[q])
        pltpu.async_copy(dst_h.at[pl.ds(off, EB)], dbufs[q], sem_m[q])
        pltpu.async_copy(et_h.at[pl.ds(off, EB)], tbufs[q], sem_m[q])

    def wait_meta(q, b):
        off = base + b * EB
        pltpu.make_async_copy(src_h.at[pl.ds(off, EB)], sbufs[q], sem_m[q]).wait()
        pltpu.make_async_copy(dst_h.at[pl.ds(off, EB)], dbufs[q], sem_m[q]).wait()
        pltpu.make_async_copy(et_h.at[pl.ds(off, EB)], tbufs[q], sem_m[q]).wait()

    def prep(q, b):
        off = base + b * EB

        def grp(g, c2):
            s16 = sbufs[q][pl.ds(g * 16, 16)]
            d16 = dbufs[q][pl.ds(g * 16, 16)]
            t16 = tbufs[q][pl.ds(g * 16, 16)]
            y1 = plsc.load_gather(ybuf, [s16])
            y2 = plsc.load_gather(ybuf, [d16])
            als = plsc.load_gather(abuf, [s16])
            gap = jnp.exp(-jnp.abs((y1 - y2).astype(jnp.float32)))
            a = als * gap
            a = jnp.where(a >= 0, a, 0.2 * a)
            ok = (t16 == 4) & ((off + g * 16 + lanes) < E)
            avbufs[q][pl.ds(g * 16, 16)] = jnp.where(ok, a, 0.0)
            sbufs[q][pl.ds(g * 16, 16)] = s16 * NC + cid
            return c2

        lax.fori_loop(0, EB // 16, grp, 0)
        pltpu.async_copy(xl2_h.at[sbufs[q]], xlbufs[q], sem_g[q])

    def wait_rows(q):
        pltpu.make_async_copy(xl2_h.at[sbufs[q]], xlbufs[q], sem_g[q]).wait()

    def scale(q):
        def edge(e, c2):
            av = plsc.load_gather(avbufs[q], [jnp.full((16,), 0, jnp.int32) + e])
            for c in range(IC2 // 16):
                sl = pl.ds(c * 16, 16)
                xlbufs[q][e, sl] = xlbufs[q][e, sl] * av
            return c2

        lax.fori_loop(0, EB, edge, 0)

    def scatter(q):
        pltpu.sync_copy(xlbufs[q], t_sh.at[dbufs[q]], add=True)

    fire_meta(0, 0)
    wait_meta(0, 0)
    prep(0, 0)
    fire_meta(1, 1)

    def pipe(i, carry):
        b = 2 * i
        wait_meta(1, b + 1)
        prep(1, b + 1)
        wait_rows(0)
        scale(0)
        scatter(0)

        @pl.when(b + 2 < NBS)
        def _():
            fire_meta(0, b + 2)

        @pl.when(b + 2 < NBS)
        def _():
            wait_meta(0, b + 2)
            prep(0, b + 2)
        wait_rows(1)
        scale(1)
        scatter(1)

        @pl.when(b + 3 < NBS)
        def _():
            fire_meta(1, b + 3)
        return carry

    lax.fori_loop(0, NBS // 2, pipe, 0)
    plsc.subcore_barrier()
    _copy_out(t_sh, out_h, cid, sid, N_DST)


@functools.partial(
    pl.kernel,
    out_type=jax.ShapeDtypeStruct((NC, NT * N_DST, 16), jnp.float32),
    mesh=_MESH,
    compiler_params=pltpu.CompilerParams(needs_layout_passes=False, use_tc_tiling_on_sc=False),
    scratch_types=[
        [pltpu.VMEM((EB,), jnp.int32)] * 2,
        [pltpu.VMEM((EB,), jnp.int32)] * 2,
        [pltpu.VMEM((EB,), jnp.int32)] * 2,
        [pltpu.VMEM((EB,), jnp.int32)] * 2,
        [pltpu.VMEM((EB,), jnp.int32)] * 2,
        [pltpu.VMEM((EB, 16), jnp.float32)] * 2,
        [pltpu.VMEM((EB, 16), jnp.float32)] * 2,
        pltpu.VMEM((H, EB), jnp.float32),
        pltpu.VMEM((EB, 16), jnp.float32),
        pltpu.VMEM_SHARED((NT * N_DST, 16), jnp.float32),
        [pltpu.SemaphoreType.DMA] * 2,
        [pltpu.SemaphoreType.DMA] * 2,
    ],
)
def _den_sc(src_h, dst_h, et_h, el_h, er_h, out_h,
            sbufs, dbufs, tbufs, gbufs, kbufs, elbufs, erbufs,
            wbuf, msgbuf, den_sh, sem_m, sem_g):
    cid = lax.axis_index("c")
    sid = lax.axis_index("s")
    wid = sid * NC + cid

    _zero_rows(msgbuf, 20, 1)
    _zero_shared(den_sh, msgbuf, sid, 97, 107, NT * N_DST)
    plsc.subcore_barrier()
    lanes = lax.iota(jnp.int32, 16)
    base = wid * EPW

    def fire_meta(q, b):
        off = base + b * EB
        pltpu.async_copy(src_h.at[pl.ds(off, EB)], sbufs[q], sem_m[q])
        pltpu.async_copy(dst_h.at[pl.ds(off, EB)], dbufs[q], sem_m[q])
        pltpu.async_copy(et_h.at[pl.ds(off, EB)], tbufs[q], sem_m[q])

    def wait_meta(q, b):
        off = base + b * EB
        pltpu.make_async_copy(src_h.at[pl.ds(off, EB)], sbufs[q], sem_m[q]).wait()
        pltpu.make_async_copy(dst_h.at[pl.ds(off, EB)], dbufs[q], sem_m[q]).wait()
        pltpu.make_async_copy(et_h.at[pl.ds(off, EB)], tbufs[q], sem_m[q]).wait()

    def prep(q):
        def mk(i, c2):
            s16 = sbufs[q][pl.ds(i * 16, 16)]
            d16 = dbufs[q][pl.ds(i * 16, 16)]
            t16 = tbufs[q][pl.ds(i * 16, 16)]
            gbufs[q][pl.ds(i * 16, 16)] = t16 * N_SRC + s16
            kbufs[q][pl.ds(i * 16, 16)] = t16 * N_DST + d16
            return c2

        lax.fori_loop(0, EB // 16, mk, 0)
        pltpu.async_copy(el_h.at[gbufs[q]], elbufs[q], sem_g[q])
        pltpu.async_copy(er_h.at[kbufs[q]], erbufs[q], sem_g[q])

    def wait_rows(q):
        pltpu.make_async_copy(el_h.at[gbufs[q]], elbufs[q], sem_g[q]).wait()
        pltpu.make_async_copy(er_h.at[kbufs[q]], erbufs[q], sem_g[q]).wait()

    def compute_scatter(q, b):
        off = base + b * EB

        def grp(g, c2):
            e16 = g * 16 + lanes
            ok = (off + e16) < E
            for h in range(H):
                hh = jnp.full((16,), h, jnp.int32)
                elh = plsc.load_gather(elbufs[q], [e16, hh])
                erh = plsc.load_gather(erbufs[q], [e16, hh])
                z = elh + erh
                z = jnp.where(z >= 0, z, 0.2 * z)
                wbuf[h, pl.ds(g * 16, 16)] = jnp.where(ok, jnp.exp(z), 0.0)
            return c2

        lax.fori_loop(0, EB // 16, grp, 0)

        def edge(e, c2):
            e0 = jnp.full((16,), 0, jnp.int32) + e
            rowv = plsc.load_gather(wbuf, [lanes & 7, e0])
            msgbuf[e, pl.ds(0, 16)] = jnp.where(lanes < 8, rowv, 0.0)
            return c2

        lax.fori_loop(0, EB, edge, 0)
        pltpu.sync_copy(msgbuf, den_sh.at[kbufs[q]], add=True)

    fire_meta(0, 0)
    wait_meta(0, 0)
    prep(0)
    fire_meta(1, 1)

    def pipe(i, carry):
        b = 2 * i
        wait_meta(1, b + 1)
        prep(1)
        wait_rows(0)
        compute_scatter(0, b)

        @pl.when(b + 2 < NB)
        def _():
            fire_meta(0, b + 2)

        @pl.when(b + 2 < NB)
        def _():
            wait_meta(0, b + 2)
            prep(0)
        wait_rows(1)
        compute_scatter(1, b + 1)

        @pl.when(b + 3 < NB)
        def _():
            fire_meta(1, b + 3)
        return carry

    lax.fori_loop(0, NB // 2, pipe, 0)
    plsc.subcore_barrier()
    _copy_out(den_sh, out_h, cid, sid, NT * N_DST)


@functools.partial(
    pl.kernel,
    out_type=jax.ShapeDtypeStruct((NC, N_DST, HH), jnp.float32),
    mesh=_MESH,
    compiler_params=pltpu.CompilerParams(needs_layout_passes=False, use_tc_tiling_on_sc=False),
    scratch_types=[
        [pltpu.VMEM((EB,), jnp.int32)] * 2,
        [pltpu.VMEM((EB,), jnp.int32)] * 2,
        [pltpu.VMEM((EB,), jnp.int32)] * 2,
        [pltpu.VMEM((EB,), jnp.int32)] * 2,
        [pltpu.VMEM((EB,), jnp.int32)] * 2,
        [pltpu.VMEM((EB,), jnp.int32)] * 2,
        [pltpu.VMEM((EB, 16), jnp.float32)] * 2,
        [pltpu.VMEM((EB, 16), jnp.float32)] * 2,
        [pltpu.VMEM((EB, 16), jnp.float32)] * 2,
        [pltpu.VMEM((EB, HH), jnp.float32)] * 2,
        pltpu.VMEM((H, EB), jnp.float32),
        pltpu.VMEM_SHARED((N_DST, HH), jnp.float32),
        [pltpu.SemaphoreType.DMA] * 2,
        [pltpu.SemaphoreType.DMA] * 2,
    ],
)
def _gat_sc(src_h, dst_h, et_h, fs_h, el_h, er_h, dr_h, out_h,
            sbufs, dbufs, tbufs, gbufs, kbufs, obufs, elbufs, erbufs, drbufs,
            fsbufs, albuf, acc_sh, sem_m, sem_g):
    cid = lax.axis_index("c")
    sid = lax.axis_index("s")
    wid = sid * NC + cid

    _zero_rows(fsbufs[0], 20, HH // 16)
    _zero_shared(acc_sh, fsbufs[0], sid, 20, 12, N_DST)
    plsc.subcore_barrier()
    lanes = lax.iota(jnp.int32, 16)
    base = sid * EPS

    def fire_meta(q, b):
        off = base + b * EB
        pltpu.async_copy(src_h.at[pl.ds(off, EB)], sbufs[q], sem_m[q])
        pltpu.async_copy(dst_h.at[pl.ds(off, EB)], dbufs[q], sem_m[q])
        pltpu.async_copy(et_h.at[pl.ds(off, EB)], tbufs[q], sem_m[q])

    def wait_meta(q, b):
        off = base + b * EB
        pltpu.make_async_copy(src_h.at[pl.ds(off, EB)], sbufs[q], sem_m[q]).wait()
        pltpu.make_async_copy(dst_h.at[pl.ds(off, EB)], dbufs[q], sem_m[q]).wait()
        pltpu.make_async_copy(et_h.at[pl.ds(off, EB)], tbufs[q], sem_m[q]).wait()

    def mk(q):
        def body(i, c2):
            s16 = sbufs[q][pl.ds(i * 16, 16)]
            d16 = dbufs[q][pl.ds(i * 16, 16)]
            t16 = tbufs[q][pl.ds(i * 16, 16)]
            fi = t16 * N_SRC + s16
            obufs[q][pl.ds(i * 16, 16)] = fi
            gbufs[q][pl.ds(i * 16, 16)] = fi + cid * (NT * N_SRC)
            kbufs[q][pl.ds(i * 16, 16)] = t16 * N_DST + d16
            return c2
        lax.fori_loop(0, EB // 16, body, 0)

    def fire_gathers(q):
        pltpu.async_copy(fs_h.at[gbufs[q]], fsbufs[q], sem_g[q])
        pltpu.async_copy(el_h.at[obufs[q]], elbufs[q], sem_g[q])
        pltpu.async_copy(er_h.at[kbufs[q]], erbufs[q], sem_g[q])
        pltpu.async_copy(dr_h.at[kbufs[q]], drbufs[q], sem_g[q])

    def wait_gathers(q):
        pltpu.make_async_copy(fs_h.at[gbufs[q]], fsbufs[q], sem_g[q]).wait()
        pltpu.make_async_copy(el_h.at[obufs[q]], elbufs[q], sem_g[q]).wait()
        pltpu.make_async_copy(er_h.at[kbufs[q]], erbufs[q], sem_g[q]).wait()
        pltpu.make_async_copy(dr_h.at[kbufs[q]], drbufs[q], sem_g[q]).wait()

    def compute(q, b):
        off = base + b * EB

        def grp(g, c2):
            e16 = g * 16 + lanes
            ok = (off + e16) < E
            for h in range(H // 2):
                hh = jnp.full((16,), h, jnp.int32) + cid * (H // 2)
                elh = plsc.load_gather(elbufs[q], [e16, hh])
                erh = plsc.load_gather(erbufs[q], [e16, hh])
                drh = plsc.load_gather(drbufs[q], [e16, hh])
                z = elh + erh
                z = jnp.where(z >= 0, z, 0.2 * z)
                albuf[h, pl.ds(g * 16, 16)] = jnp.where(ok, jnp.exp(z) * drh, 0.0)
            return c2

        lax.fori_loop(0, EB // 16, grp, 0)

        def edge(e, c2):
            e0 = jnp.full((16,), 0, jnp.int32) + e
            for h in range(H // 2):
                av = plsc.load_gather(albuf, [jnp.full((16,), h, jnp.int32), e0])
                for c in range(DH // 16):
                    sl = pl.ds(h * DH + c * 16, 16)
                    fsbufs[q][e, sl] = fsbufs[q][e, sl] * av
            return c2

        lax.fori_loop(0, EB, edge, 0)

    def scatter(q):
        pltpu.sync_copy(fsbufs[q], acc_sh.at[dbufs[q]], add=True)

    fire_meta(0, 0)
    wait_meta(0, 0)
    mk(0)
    fire_gathers(0)
    fire_meta(1, 1)

    def pipe(i, carry):
        b = 2 * i

        wait_meta(1, b + 1)
        mk(1)
        fire_gathers(1)
        wait_gathers(0)
        compute(0, b)
        scatter(0)

        @pl.when(b + 2 < NBS)
        def _():
            fire_meta(0, b + 2)

        @pl.when(b + 2 < NBS)
        def _():
            wait_meta(0, b + 2)
            mk(0)
            fire_gathers(0)
        wait_gathers(1)
        compute(1, b + 1)
        scatter(1)

        @pl.when(b + 3 < NBS)
        def _():
            fire_meta(1, b + 3)
        return carry

    lax.fori_loop(0, NBS // 2, pipe, 0)
    plsc.subcore_barrier()
    _copy_out(acc_sh, out_h, cid, sid, N_DST)


def kernel(x, edge_index, etype, years, n_dst, W_skip, b_skip, W_gat,
           attn_l, attn_r, b_gat, att_t, W_t, b_t, bn1_g, bn1_b,
           W_m1, b_m1, bnm_g, bnm_b, W_m2, b_m2):
    f32 = jnp.float32
    i32 = jnp.int32
    src = edge_index[0].astype(i32)
    dst = edge_index[1].astype(i32)
    et = etype.astype(i32)
    yrs = years.astype(i32)

    pad = EP - E
    srcp = jnp.concatenate([src, jnp.zeros((pad,), i32)])
    dstp = jnp.concatenate([dst, jnp.zeros((pad,), i32)])
    etp = jnp.concatenate([et, jnp.zeros((pad,), i32)])

    Wg4 = W_gat.reshape(NT, IN_CH, H, DH)
    WL = jnp.einsum("jchd,jhd->jch", Wg4, attn_l)
    WR = jnp.einsum("jchd,jhd->jch", Wg4, attn_r)
    zpad = jnp.zeros((NT, IN_CH, 8), f32)
    WLpad = jnp.concatenate([WL, zpad], axis=2)
    WRpad = jnp.concatenate([WR, zpad], axis=2)
    wt_att = W_t @ att_t[0]
    b_att = jnp.dot(b_t, att_t[0])
    delta = (jnp.asarray(n_dst) - N_DST).astype(f32)
    bskip_eff = b_skip + delta + b_gat.sum(0)
    c_bn = 1.0 / jnp.sqrt(1.0 + 1e-5)

    AUXA = 384
    WA = jnp.concatenate([W_t, wt_att[:, None], jnp.zeros((IN_CH, AUXA - 257), f32)], axis=1)
    bA = jnp.concatenate([b_t, b_att[None], jnp.zeros((AUXA - 257,), f32)])
    auxA = _matmul_bias(x, WA, bA, 2000)
    xl = auxA[:, :IN_CH]
    alt = auxA[:, IN_CH]

    xskip = _matmul_bias(x[:N_DST], W_skip, bskip_eff, N_DST)

    fs = _fs_half_mm(x, W_gat.reshape(NT, IN_CH, NC, HH).transpose(2, 0, 1, 3), 2000)
    fs2 = fs.reshape(NC * NT * N_SRC, HH)
    el = _etype_mm(x, WLpad, 2000)
    elflat = el.reshape(NT * N_SRC, 16)
    er03 = _etype_mm(x[:N_DST], WRpad[:4], N_DST)

    xl2 = xl.reshape(N_SRC * NC, IC2)
    tparts = _temporal_sc(srcp, dstp, etp, yrs, alt, xl2)
    er4 = _matmul_bias(
        jnp.concatenate([tparts[0], tparts[1]], axis=1),
        WRpad[4], jnp.zeros((16,), f32), N_DST)
    erflat = jnp.concatenate([er03, er4[None]], axis=0).reshape(NT * N_DST, 16)

    denparts = _den_sc(srcp, dstp, etp, elflat, erflat)
    denr = _denr(denparts)

    accparts = _gat_sc(srcp, dstp, etp, fs2, elflat, erflat, denr)

    return _final(accparts, xskip,
                  bn1_g * c_bn, bn1_b, W_m1, b_m1,
                  bnm_g * c_bn, bnm_b, W_m2, b_m2)
